# baseline (speedup 1.0000x reference)
import functools

import jax
import jax.numpy as jnp
import numpy as np
from jax import lax
from jax.experimental import pallas as pl
from jax.experimental.pallas import tpu as pltpu
from jax.experimental.pallas import tpu_sc as plsc

D_MODEL = 512
MAX_LEN = 100
BATCH = 1024
TOKENS = BATCH * MAX_LEN

NC = 2
NS = 16
L = 16
NW = NC * NS
CHUNK = BATCH // NW
NBUF = 4
AHEAD = 2
VLANES = D_MODEL // L
RUNROLL = 4


def _pos_encoding_np():
    even_i = np.arange(0, D_MODEL, 2, dtype=np.float32)
    denominator = np.power(10000.0, even_i / D_MODEL)
    position = np.arange(MAX_LEN, dtype=np.float32).reshape(MAX_LEN, 1)
    even_pe = np.sin(position / denominator)
    odd_pe = np.cos(position / denominator)
    return np.stack([even_pe, odd_pe], axis=2).reshape(MAX_LEN, D_MODEL)


_PE = _pos_encoding_np()


def _make_kernel():
    mesh = plsc.VectorSubcoreMesh(core_axis_name="c", subcore_axis_name="s")

    @functools.partial(
        pl.kernel,
        mesh=mesh,
        out_type=jax.ShapeDtypeStruct((TOKENS, D_MODEL), jnp.float32),
        scratch_types=[
            pltpu.VMEM((MAX_LEN * CHUNK,), jnp.int32),
            pltpu.VMEM((MAX_LEN, D_MODEL), jnp.float32),
        ]
        + [pltpu.VMEM((CHUNK, D_MODEL), jnp.float32) for _ in range(NBUF)]
        + [pltpu.SemaphoreType.DMA, pltpu.SemaphoreType.DMA],
    )
    def k(idx_hbm, table_hbm, pe_hbm, out_hbm, idx_v, pe_v,
          rows0, rows1, rows2, rows3, gsem, wsem):
        bufs = (rows0, rows1, rows2, rows3)
        wid = lax.axis_index("s") * NC + lax.axis_index("c")
        b0 = wid * CHUNK
        pltpu.sync_copy(pe_hbm, pe_v)
        pltpu.sync_copy(idx_hbm.at[pl.ds(wid * MAX_LEN * CHUNK, MAX_LEN * CHUNK)],
                        idx_v)

        def gather(c, buf):
            pltpu.async_copy(
                table_hbm.at[idx_v.at[pl.ds(c * CHUNK, CHUNK)]], buf, gsem
            )

        def drain_gather(buf):
            pltpu.make_async_copy(table_hbm.at[pl.ds(0, CHUNK)], buf, gsem).wait()

        def store(c, buf):
            pltpu.async_copy(buf, out_hbm.at[pl.ds(c * BATCH + b0, CHUNK)], wsem)

        def drain_store(buf):
            pltpu.make_async_copy(buf, out_hbm.at[pl.ds(0, CHUNK)], wsem).wait()

        def add_pe(c, buf):
            pe_regs = tuple(pe_v[c, pl.ds(cc * L, L)] for cc in range(VLANES))

            def add_rows(rr, regs):
                for u in range(RUNROLL):
                    r = rr * RUNROLL + u
                    for cc in range(VLANES):
                        sl = pl.ds(cc * L, L)
                        buf[r, sl] = buf[r, sl] + regs[cc]
                return regs

            lax.fori_loop(0, CHUNK // RUNROLL, add_rows, pe_regs)

        for c in range(AHEAD):
            gather(c, bufs[c])

        def body(j, carry):
            for kk in range(NBUF):
                c = NBUF * j + kk
                drain_gather(bufs[kk])
                add_pe(c, bufs[kk])
                store(c, bufs[kk])

                @pl.when(c >= AHEAD)
                def _():
                    drain_store(bufs[(kk + AHEAD) % NBUF])

                @pl.when(c < MAX_LEN - AHEAD)
                def _():
                    gather(c + AHEAD, bufs[(kk + AHEAD) % NBUF])
            return carry

        lax.fori_loop(0, MAX_LEN // NBUF, body, None)
        for kk in range(AHEAD):
            drain_store(bufs[(MAX_LEN - AHEAD + kk) % NBUF])

    return k


def kernel(indices, table):
    idx_r = (indices.astype(jnp.int32)
             .T.reshape(MAX_LEN, NW, CHUNK)
             .transpose(1, 0, 2)
             .reshape(TOKENS))
    pe = jnp.asarray(_PE)
    out = _make_kernel()(idx_r, table, pe)
    return out.reshape(MAX_LEN, BATCH, D_MODEL).transpose(1, 0, 2)


# speedup vs baseline: 1.0006x; 1.0006x over previous
import functools

import jax
import jax.numpy as jnp
import numpy as np
from jax import lax
from jax.experimental import pallas as pl
from jax.experimental.pallas import tpu as pltpu
from jax.experimental.pallas import tpu_sc as plsc

D_MODEL = 512
MAX_LEN = 100
BATCH = 1024
TOKENS = BATCH * MAX_LEN

NC = 2
NS = 16
L = 16
NW = NC * NS
CHUNK = BATCH // NW
NBUF = 4
AHEAD = 2
VLANES = D_MODEL // L
RUNROLL = 4


def _pos_encoding_np():
    even_i = np.arange(0, D_MODEL, 2, dtype=np.float32)
    denominator = np.power(10000.0, even_i / D_MODEL)
    position = np.arange(MAX_LEN, dtype=np.float32).reshape(MAX_LEN, 1)
    even_pe = np.sin(position / denominator)
    odd_pe = np.cos(position / denominator)
    return np.stack([even_pe, odd_pe], axis=2).reshape(MAX_LEN, D_MODEL)


_PE = _pos_encoding_np()


def _make_kernel():
    mesh = plsc.VectorSubcoreMesh(core_axis_name="c", subcore_axis_name="s")

    @functools.partial(
        pl.kernel,
        mesh=mesh,
        out_type=jax.ShapeDtypeStruct((TOKENS, D_MODEL), jnp.float32),
        scratch_types=[
            pltpu.VMEM((MAX_LEN * CHUNK,), jnp.int32),
            pltpu.VMEM((MAX_LEN, D_MODEL), jnp.float32),
        ]
        + [pltpu.VMEM((CHUNK, D_MODEL), jnp.float32) for _ in range(NBUF)]
        + [pltpu.SemaphoreType.DMA, pltpu.SemaphoreType.DMA],
    )
    def k(idx_hbm, table_hbm, pe_hbm, out_hbm, idx_v, pe_v,
          rows0, rows1, rows2, rows3, gsem, wsem):
        bufs = (rows0, rows1, rows2, rows3)
        wid = lax.axis_index("s") * NC + lax.axis_index("c")
        b0 = wid * CHUNK
        pltpu.sync_copy(pe_hbm, pe_v)
        pltpu.sync_copy(idx_hbm.at[pl.ds(wid * MAX_LEN * CHUNK, MAX_LEN * CHUNK)],
                        idx_v)

        def gather(c, buf):
            pltpu.async_copy(
                table_hbm.at[idx_v.at[pl.ds(c * CHUNK, CHUNK)]], buf, gsem
            )

        def drain_gather(buf):
            pltpu.make_async_copy(table_hbm.at[pl.ds(0, CHUNK)], buf, gsem).wait()

        def store(c, buf):
            pltpu.async_copy(buf, out_hbm.at[pl.ds(c * BATCH + b0, CHUNK)], wsem)

        def drain_store(buf):
            pltpu.make_async_copy(buf, out_hbm.at[pl.ds(0, CHUNK)], wsem).wait()

        def add_pe(c, buf):
            pe_regs = tuple(pe_v[c, pl.ds(cc * L, L)] for cc in range(VLANES))

            def add_rows(rr, regs):
                for u in range(RUNROLL):
                    r = rr * RUNROLL + u
                    for cc in range(VLANES):
                        sl = pl.ds(cc * L, L)
                        buf[r, sl] = buf[r, sl] + regs[cc]
                return regs

            lax.fori_loop(0, CHUNK // RUNROLL, add_rows, pe_regs)

        for c in range(AHEAD):
            gather(c, bufs[c])

        def body(j, carry):
            for kk in range(NBUF):
                c = NBUF * j + kk
                drain_gather(bufs[kk])

                @pl.when(c >= AHEAD)
                def _():
                    drain_store(bufs[(kk + AHEAD) % NBUF])

                @pl.when(c < MAX_LEN - AHEAD)
                def _():
                    gather(c + AHEAD, bufs[(kk + AHEAD) % NBUF])

                add_pe(c, bufs[kk])
                store(c, bufs[kk])
            return carry

        lax.fori_loop(0, MAX_LEN // NBUF, body, None)
        for kk in range(AHEAD):
            drain_store(bufs[(MAX_LEN - AHEAD + kk) % NBUF])

    return k


def kernel(indices, table):
    idx_r = (indices.astype(jnp.int32)
             .T.reshape(MAX_LEN, NW, CHUNK)
             .transpose(1, 0, 2)
             .reshape(TOKENS))
    pe = jnp.asarray(_PE)
    out = _make_kernel()(idx_r, table, pe)
    return out.reshape(MAX_LEN, BATCH, D_MODEL).transpose(1, 0, 2)


# speedup vs baseline: 1.7834x; 1.7823x over previous
import functools

import jax
import jax.numpy as jnp
import numpy as np
from jax import lax
from jax.experimental import pallas as pl
from jax.experimental.pallas import tpu as pltpu
from jax.experimental.pallas import tpu_sc as plsc

D_MODEL = 512
MAX_LEN = 100
BATCH = 1024
TOKENS = BATCH * MAX_LEN

NC = 2
NS = 16
L = 16
NW = NC * NS
CHUNK = BATCH // NW
NBUF = 4
AHEAD = 2
VLANES = D_MODEL // L
RUNROLL = 1


def _pos_encoding_np():
    even_i = np.arange(0, D_MODEL, 2, dtype=np.float32)
    denominator = np.power(10000.0, even_i / D_MODEL)
    position = np.arange(MAX_LEN, dtype=np.float32).reshape(MAX_LEN, 1)
    even_pe = np.sin(position / denominator)
    odd_pe = np.cos(position / denominator)
    return np.stack([even_pe, odd_pe], axis=2).reshape(MAX_LEN, D_MODEL)


_PE = _pos_encoding_np()


def _make_kernel():
    mesh = plsc.VectorSubcoreMesh(core_axis_name="c", subcore_axis_name="s")

    @functools.partial(
        pl.kernel,
        mesh=mesh,
        out_type=jax.ShapeDtypeStruct((TOKENS, D_MODEL), jnp.float32),
        scratch_types=[
            pltpu.VMEM((MAX_LEN * CHUNK,), jnp.int32),
            pltpu.VMEM((MAX_LEN, D_MODEL), jnp.float32),
        ]
        + [pltpu.VMEM((CHUNK, D_MODEL), jnp.float32) for _ in range(NBUF)]
        + [pltpu.SemaphoreType.DMA, pltpu.SemaphoreType.DMA],
    )
    def k(idx_hbm, table_hbm, pe_hbm, out_hbm, idx_v, pe_v,
          rows0, rows1, rows2, rows3, gsem, wsem):
        bufs = (rows0, rows1, rows2, rows3)
        wid = lax.axis_index("s") * NC + lax.axis_index("c")
        b0 = wid * CHUNK
        pltpu.sync_copy(pe_hbm, pe_v)
        pltpu.sync_copy(idx_hbm.at[pl.ds(wid * MAX_LEN * CHUNK, MAX_LEN * CHUNK)],
                        idx_v)

        def gather(c, buf):
            pltpu.async_copy(
                table_hbm.at[idx_v.at[pl.ds(c * CHUNK, CHUNK)]], buf, gsem
            )

        def drain_gather(buf):
            pltpu.make_async_copy(table_hbm.at[pl.ds(0, CHUNK)], buf, gsem).wait()

        def store(c, buf):
            pltpu.async_copy(buf, out_hbm.at[pl.ds(c * BATCH + b0, CHUNK)], wsem)

        def drain_store(buf):
            pltpu.make_async_copy(buf, out_hbm.at[pl.ds(0, CHUNK)], wsem).wait()

        def add_pe(c, buf):
            pe_regs = tuple(pe_v[c, pl.ds(cc * L, L)] for cc in range(VLANES))

            def add_rows(rr, regs):
                for u in range(RUNROLL):
                    r = rr * RUNROLL + u
                    for cc in range(VLANES):
                        sl = pl.ds(cc * L, L)
                        buf[r, sl] = buf[r, sl] + regs[cc]
                return regs

            lax.fori_loop(0, CHUNK // RUNROLL, add_rows, pe_regs)

        for c in range(AHEAD):
            gather(c, bufs[c])

        def body(j, carry):
            for kk in range(NBUF):
                c = NBUF * j + kk
                drain_gather(bufs[kk])

                @pl.when(c >= AHEAD)
                def _():
                    drain_store(bufs[(kk + AHEAD) % NBUF])

                @pl.when(c < MAX_LEN - AHEAD)
                def _():
                    gather(c + AHEAD, bufs[(kk + AHEAD) % NBUF])

                add_pe(c, bufs[kk])
                store(c, bufs[kk])
            return carry

        lax.fori_loop(0, MAX_LEN // NBUF, body, None)
        for kk in range(AHEAD):
            drain_store(bufs[(MAX_LEN - AHEAD + kk) % NBUF])

    return k


def kernel(indices, table):
    idx_r = (indices.astype(jnp.int32)
             .T.reshape(MAX_LEN, NW, CHUNK)
             .transpose(1, 0, 2)
             .reshape(TOKENS))
    pe = jnp.asarray(_PE)
    out = _make_kernel()(idx_r, table, pe)
    return out.reshape(MAX_LEN, BATCH, D_MODEL).transpose(1, 0, 2)
